# baseline (speedup 1.0000x reference)
import functools

import jax
import jax.numpy as jnp
from jax import lax
from jax.experimental import pallas as pl
from jax.experimental.pallas import tpu as pltpu
from jax.experimental.pallas import tpu_sc as plsc

_BN_EPS = 1e-5
_NC, _NS = 2, 16
_NW = _NC * _NS
_CHUNK = 64
_NSLOT = 5
_NPAD = 112
_G = 64
_NB = 10


def _sc_aggregate(h, src3, dst3):
    N, D = h.shape
    CH = src3.shape[1]
    mesh = plsc.VectorSubcoreMesh(core_axis_name="c", subcore_axis_name="s")

    NSLOT = _NSLOT
    assert CH % NSLOT == 0 and CH >= 2 * NSLOT

    @functools.partial(
        pl.kernel,
        out_type=jax.ShapeDtypeStruct((_NC, N + _NPAD, D), jnp.float32),
        mesh=mesh,
        scratch_types=[
            [pltpu.VMEM((_CHUNK, D), jnp.float32)] * NSLOT,
            [pltpu.VMEM((1, _CHUNK), jnp.int32)] * NSLOT,
            [pltpu.VMEM((1, _CHUNK), jnp.int32)] * NSLOT,
            pltpu.VMEM_SHARED((N + _NPAD, D), jnp.float32),
            [pltpu.SemaphoreType.DMA] * NSLOT,
            [pltpu.SemaphoreType.DMA] * NSLOT,
            [pltpu.SemaphoreType.DMA] * NSLOT,
            [pltpu.SemaphoreType.DMA] * NSLOT,
        ],
    )
    def agg_kernel(h_hbm, src_hbm, dst_hbm, out_hbm,
                   rows, sidx, didx, agg_sh, gsems, ssems, sisems, disems):
        c = lax.axis_index("c")
        s = lax.axis_index("s")
        wid = c * _NS + s

        def zrow(r, carry):
            for q in range(D // 16):
                rows[0][r, pl.ds(q * 16, 16)] = jnp.zeros((16,), jnp.float32)
            return carry

        lax.fori_loop(0, _CHUNK, zrow, 0)
        rz = (N + _NPAD) // _NS
        base_r = s * rz
        nfull, rem = rz // _CHUNK, rz % _CHUNK
        for q in range(nfull):
            pltpu.sync_copy(rows[0], agg_sh.at[pl.ds(base_r + q * _CHUNK, _CHUNK)])
        if rem:
            pltpu.sync_copy(rows[0].at[pl.ds(0, rem)],
                            agg_sh.at[pl.ds(base_r + nfull * _CHUNK, rem)])
        plsc.subcore_barrier()

        def si_start(k, sl):
            pltpu.async_copy(src_hbm.at[wid, pl.ds(k, 1)], sidx[sl], sisems[sl])

        def si_wait(k, sl):
            pltpu.make_async_copy(src_hbm.at[wid, pl.ds(k, 1)], sidx[sl], sisems[sl]).wait()

        def di_start(k, sl):
            pltpu.async_copy(dst_hbm.at[wid, pl.ds(k, 1)], didx[sl], disems[sl])

        def di_wait(k, sl):
            pltpu.make_async_copy(dst_hbm.at[wid, pl.ds(k, 1)], didx[sl], disems[sl]).wait()

        def g_start(k, sl):
            pltpu.async_copy(h_hbm.at[sidx[sl].at[0]], rows[sl], gsems[sl])

        def g_wait(k, sl):
            pltpu.make_async_copy(h_hbm.at[sidx[sl].at[0]], rows[sl], gsems[sl]).wait()

        def s_start(k, sl):
            pltpu.async_copy(rows[sl], agg_sh.at[didx[sl].at[0]], ssems[sl], add=True)

        def s_wait(k, sl):
            pltpu.make_async_copy(rows[sl], agg_sh.at[didx[sl].at[0]], ssems[sl]).wait()

        for k in range(NSLOT):
            si_start(k, k)
            di_start(k, k)
        for k in range(NSLOT - 1):
            si_wait(k, k)
            g_start(k, k)

        def body(jj, carry):
            j0 = jj * NSLOT
            for ph in range(NSLOT):
                j = j0 + ph
                g_wait(j, ph)

                @pl.when(j + NSLOT < CH)
                def _prefetch_src(j=j, ph=ph):
                    si_start(j + NSLOT, ph)

                di_wait(j, ph)
                s_start(j, ph)
                tgt = (ph + NSLOT - 1) % NSLOT

                @pl.when(j + NSLOT - 1 < CH)
                def _start_next(j=j, tgt=tgt):
                    @pl.when(j >= 1)
                    def _free_slot():
                        s_wait(j - 1, tgt)

                    si_wait(j + NSLOT - 1, tgt)
                    g_start(j + NSLOT - 1, tgt)

                    @pl.when(j >= 1)
                    def _prefetch_dst():
                        di_start(j + NSLOT - 1, tgt)

            return carry

        lax.fori_loop(0, CH // NSLOT, body, 0)
        for j in range(CH - NSLOT, CH):
            s_wait(j, j % NSLOT)
        plsc.subcore_barrier()
        pltpu.sync_copy(agg_sh.at[pl.ds(s * rz, rz)], out_hbm.at[c, pl.ds(s * rz, rz)])

    return agg_kernel(h, src3, dst3)


def _tc_layer(h, aggs, p, has_base, final_args=None):
    N, D = h.shape
    BR = N // _NB
    inv_n = 1.0 / N
    is_final = final_args is not None

    def body(*refs):
        if is_final:
            (h_ref, a_ref, W1_ref, b1_ref, W2_ref, b2_ref, g_ref, be_ref,
             lw_ref, lb_ref, bt_ref, out_ref,
             u2_all, h_all, st_ref, cnt_ref) = refs
        else:
            (h_ref, a_ref, W1_ref, b1_ref, W2_ref, b2_ref, g_ref, be_ref,
             out_ref, u2_all, h_all, st_ref) = refs
        i = pl.program_id(0)

        @pl.when(i < _NB)
        def _compute():
            u = h_ref[...] + a_ref[0] + a_ref[1]
            r1 = jnp.maximum(
                jnp.dot(u, W1_ref[...], preferred_element_type=jnp.float32)
                + b1_ref[...], 0.0)
            u2 = jnp.maximum(
                jnp.dot(r1, W2_ref[...], preferred_element_type=jnp.float32)
                + b2_ref[...], 0.0)
            u2_all[pl.ds(i * BR, BR), :] = u2
            if has_base:
                h_all[pl.ds(i * BR, BR), :] = h_ref[...]

            @pl.when(i == 0)
            def _init():
                st_ref[...] = jnp.zeros_like(st_ref)

            st_ref[0:1, :] += jnp.sum(u2, axis=0, keepdims=True)
            st_ref[1:2, :] += jnp.sum(u2 * u2, axis=0, keepdims=True)

        @pl.when(i >= _NB)
        def _normalize():
            blk = i - _NB
            u2 = u2_all[pl.ds(blk * BR, BR), :]
            mean = st_ref[0:1, :] * inv_n
            var = st_ref[1:2, :] * inv_n - mean * mean
            scale = g_ref[...] * lax.rsqrt(var + _BN_EPS)
            r = (u2 - mean) * scale + be_ref[...]
            if has_base:
                r = h_all[pl.ds(blk * BR, BR), :] + r
            if not is_final:
                out_ref[...] = r
            else:
                t = jnp.tanh(
                    jnp.dot(r, lw_ref[...], preferred_element_type=jnp.float32)
                    + lb_ref[...])
                seg = bt_ref[0]
                gids = lax.broadcasted_iota(jnp.int32, (_G, BR), 0)
                oh = (gids == seg).astype(jnp.float32)

                @pl.when(i == _NB)
                def _init_pool():
                    out_ref[...] = jnp.zeros_like(out_ref)
                    cnt_ref[...] = jnp.zeros_like(cnt_ref)

                out_ref[...] += jnp.dot(oh, t, preferred_element_type=jnp.float32)
                cnt_ref[...] += jnp.sum(oh, axis=1, keepdims=True)

                @pl.when(i == 2 * _NB - 1)
                def _fin():
                    out_ref[...] = out_ref[...] / jnp.maximum(cnt_ref[...], 1.0)

    frozen = lambda i: (jnp.minimum(i, _NB - 1), 0)
    const2 = lambda i: (0, 0)
    in_specs = [
        pl.BlockSpec((BR, D), frozen),
        pl.BlockSpec((_NC, BR, D), lambda i: (0, jnp.minimum(i, _NB - 1), 0)),
        pl.BlockSpec((D, D), const2),
        pl.BlockSpec((1, D), const2),
        pl.BlockSpec((D, D), const2),
        pl.BlockSpec((1, D), const2),
        pl.BlockSpec((1, D), const2),
        pl.BlockSpec((1, D), const2),
    ]
    args = [h, aggs, p['W1'], p['b1'].reshape(1, D), p['W2'], p['b2'].reshape(1, D),
            p['gamma'].reshape(1, D), p['beta'].reshape(1, D)]
    scratch = [
        pltpu.VMEM((N, D), jnp.float32),
        pltpu.VMEM((N if has_base else 8, D), jnp.float32),
        pltpu.VMEM((8, D), jnp.float32),
    ]
    if is_final:
        lin_W, lin_b, batch3 = final_args
        in_specs += [
            pl.BlockSpec((D, D), const2),
            pl.BlockSpec((1, D), const2),
            pl.BlockSpec((1, 1, BR), lambda i: (jnp.maximum(i - _NB, 0), 0, 0)),
        ]
        args += [lin_W, lin_b.reshape(1, D), batch3]
        out_spec = pl.BlockSpec((_G, D), const2)
        out_shape = jax.ShapeDtypeStruct((_G, D), jnp.float32)
        scratch.append(pltpu.VMEM((_G, D), jnp.float32))
    else:
        out_spec = pl.BlockSpec((BR, D), lambda i: (jnp.maximum(i - _NB, 0), 0))
        out_shape = jax.ShapeDtypeStruct((N, D), jnp.float32)

    return pl.pallas_call(
        body,
        grid=(2 * _NB,),
        in_specs=in_specs,
        out_specs=out_spec,
        out_shape=out_shape,
        scratch_shapes=scratch,
    )(*args)


def kernel(x, edge_index, batch, params):
    N, D = x.shape
    E = edge_index.shape[1]

    step = _CHUNK * _NSLOT
    per_w = -(-E // (_NW * step)) * step
    e_pad = _NW * per_w
    pad = e_pad - E
    perm = jnp.argsort(edge_index[1])
    edge_index = edge_index[:, perm]
    ar = jnp.arange(pad, dtype=jnp.int32)
    fill_src = (ar * 97) % N
    fill_dst = N + (ar % _NPAD)
    src3 = jnp.concatenate([edge_index[0], fill_src]).reshape(_NW, per_w // _CHUNK, _CHUNK)
    dst3 = jnp.concatenate([edge_index[1], fill_dst]).reshape(_NW, per_w // _CHUNK, _CHUNK)
    batch3 = batch.reshape(_NB, 1, N // _NB)

    h = x
    convs = [params['conv1']] + list(params['convs'])
    for li, p in enumerate(convs):
        aggs = _sc_aggregate(h, src3, dst3)
        fin = (params['lin_W'], params['lin_b'], batch3) if li == len(convs) - 1 else None
        h = _tc_layer(h, aggs, p, has_base=li > 0, final_args=fin)
    return h


# speedup vs baseline: 1.6838x; 1.6838x over previous
import functools

import jax
import jax.numpy as jnp
from jax import lax
from jax.experimental import pallas as pl
from jax.experimental.pallas import tpu as pltpu
from jax.experimental.pallas import tpu_sc as plsc

_BN_EPS = 1e-5
_NC, _NS = 2, 16
_NW = _NC * _NS
_CHUNK = 64
_NSLOT = 5
_NPAD = 112
_G = 64
_NB = 10


def _sc_aggregate(h, src3, dst3):
    N, D = h.shape
    CH = src3.shape[1]
    mesh = plsc.VectorSubcoreMesh(core_axis_name="c", subcore_axis_name="s")

    NSLOT = _NSLOT
    assert CH % NSLOT == 0 and CH >= 2 * NSLOT

    @functools.partial(
        pl.kernel,
        out_type=jax.ShapeDtypeStruct((_NC, N + _NPAD, D), jnp.float32),
        mesh=mesh,
        scratch_types=[
            [pltpu.VMEM((_CHUNK, D), jnp.float32)] * NSLOT,
            [pltpu.VMEM((1, _CHUNK), jnp.int32)] * NSLOT,
            [pltpu.VMEM((1, _CHUNK), jnp.int32)] * NSLOT,
            pltpu.VMEM_SHARED((N + _NPAD, D), jnp.float32),
            [pltpu.SemaphoreType.DMA] * NSLOT,
            [pltpu.SemaphoreType.DMA] * NSLOT,
            [pltpu.SemaphoreType.DMA] * NSLOT,
            [pltpu.SemaphoreType.DMA] * NSLOT,
        ],
    )
    def agg_kernel(h_hbm, src_hbm, dst_hbm, out_hbm,
                   rows, sidx, didx, agg_sh, gsems, ssems, sisems, disems):
        c = lax.axis_index("c")
        s = lax.axis_index("s")
        wid = c * _NS + s

        def zrow(r, carry):
            for q in range(D // 16):
                rows[0][r, pl.ds(q * 16, 16)] = jnp.zeros((16,), jnp.float32)
            return carry

        lax.fori_loop(0, _CHUNK, zrow, 0)
        rz = (N + _NPAD) // _NS
        base_r = s * rz
        nfull, rem = rz // _CHUNK, rz % _CHUNK
        for q in range(nfull):
            pltpu.sync_copy(rows[0], agg_sh.at[pl.ds(base_r + q * _CHUNK, _CHUNK)])
        if rem:
            pltpu.sync_copy(rows[0].at[pl.ds(0, rem)],
                            agg_sh.at[pl.ds(base_r + nfull * _CHUNK, rem)])
        plsc.subcore_barrier()

        def si_start(k, sl):
            pltpu.async_copy(src_hbm.at[wid, pl.ds(k, 1)], sidx[sl], sisems[sl])

        def si_wait(k, sl):
            pltpu.make_async_copy(src_hbm.at[wid, pl.ds(k, 1)], sidx[sl], sisems[sl]).wait()

        def di_start(k, sl):
            pltpu.async_copy(dst_hbm.at[wid, pl.ds(k, 1)], didx[sl], disems[sl])

        def di_wait(k, sl):
            pltpu.make_async_copy(dst_hbm.at[wid, pl.ds(k, 1)], didx[sl], disems[sl]).wait()

        def g_start(k, sl):
            pltpu.async_copy(h_hbm.at[sidx[sl].at[0]], rows[sl], gsems[sl])

        def g_wait(k, sl):
            pltpu.make_async_copy(h_hbm.at[sidx[sl].at[0]], rows[sl], gsems[sl]).wait()

        def s_start(k, sl):
            pltpu.async_copy(rows[sl], agg_sh.at[didx[sl].at[0]], ssems[sl], add=True)

        def s_wait(k, sl):
            pltpu.make_async_copy(rows[sl], agg_sh.at[didx[sl].at[0]], ssems[sl]).wait()

        for k in range(NSLOT):
            si_start(k, k)
            di_start(k, k)
        for k in range(NSLOT - 1):
            si_wait(k, k)
            g_start(k, k)

        def body(jj, carry):
            j0 = jj * NSLOT
            for ph in range(NSLOT):
                j = j0 + ph
                g_wait(j, ph)

                @pl.when(j + NSLOT < CH)
                def _prefetch_src(j=j, ph=ph):
                    si_start(j + NSLOT, ph)

                di_wait(j, ph)
                s_start(j, ph)
                tgt = (ph + NSLOT - 1) % NSLOT

                @pl.when(j + NSLOT - 1 < CH)
                def _start_next(j=j, tgt=tgt):
                    @pl.when(j >= 1)
                    def _free_slot():
                        s_wait(j - 1, tgt)

                    si_wait(j + NSLOT - 1, tgt)
                    g_start(j + NSLOT - 1, tgt)

                    @pl.when(j >= 1)
                    def _prefetch_dst():
                        di_start(j + NSLOT - 1, tgt)

            return carry

        lax.fori_loop(0, CH // NSLOT, body, 0)
        for j in range(CH - NSLOT, CH):
            s_wait(j, j % NSLOT)
        plsc.subcore_barrier()
        pltpu.sync_copy(agg_sh.at[pl.ds(s * rz, rz)], out_hbm.at[c, pl.ds(s * rz, rz)])

    return agg_kernel(h, src3, dst3)


def _tc_layer(h, aggs, p, has_base, final_args=None):
    N, D = h.shape
    BR = N // _NB
    inv_n = 1.0 / N
    is_final = final_args is not None

    def body(*refs):
        if is_final:
            (h_ref, a_ref, W1_ref, b1_ref, W2_ref, b2_ref, g_ref, be_ref,
             lw_ref, lb_ref, bt_ref, out_ref,
             u2_all, h_all, st_ref, cnt_ref) = refs
        else:
            (h_ref, a_ref, W1_ref, b1_ref, W2_ref, b2_ref, g_ref, be_ref,
             out_ref, u2_all, h_all, st_ref) = refs
        i = pl.program_id(0)

        @pl.when(i < _NB)
        def _compute():
            u = h_ref[...] + a_ref[0] + a_ref[1]
            r1 = jnp.maximum(
                jnp.dot(u.astype(jnp.bfloat16), W1_ref[...].astype(jnp.bfloat16),
                        preferred_element_type=jnp.float32)
                + b1_ref[...], 0.0)
            u2 = jnp.maximum(
                jnp.dot(r1.astype(jnp.bfloat16), W2_ref[...].astype(jnp.bfloat16),
                        preferred_element_type=jnp.float32)
                + b2_ref[...], 0.0)
            u2_all[pl.ds(i * BR, BR), :] = u2
            if has_base:
                h_all[pl.ds(i * BR, BR), :] = h_ref[...]

            @pl.when(i == 0)
            def _init():
                st_ref[...] = jnp.zeros_like(st_ref)

            st_ref[0:1, :] += jnp.sum(u2, axis=0, keepdims=True)
            st_ref[1:2, :] += jnp.sum(u2 * u2, axis=0, keepdims=True)

        @pl.when(i >= _NB)
        def _normalize():
            blk = i - _NB
            u2 = u2_all[pl.ds(blk * BR, BR), :]
            mean = st_ref[0:1, :] * inv_n
            var = st_ref[1:2, :] * inv_n - mean * mean
            scale = g_ref[...] * lax.rsqrt(var + _BN_EPS)
            r = (u2 - mean) * scale + be_ref[...]
            if has_base:
                r = h_all[pl.ds(blk * BR, BR), :] + r
            if not is_final:
                out_ref[...] = r
            else:
                t = jnp.tanh(
                    jnp.dot(r, lw_ref[...], preferred_element_type=jnp.float32)
                    + lb_ref[...])
                seg = bt_ref[0]
                gids = lax.broadcasted_iota(jnp.int32, (_G, BR), 0)
                oh = (gids == seg).astype(jnp.float32)

                @pl.when(i == _NB)
                def _init_pool():
                    out_ref[...] = jnp.zeros_like(out_ref)
                    cnt_ref[...] = jnp.zeros_like(cnt_ref)

                out_ref[...] += jnp.dot(oh, t, preferred_element_type=jnp.float32)
                cnt_ref[...] += jnp.sum(oh, axis=1, keepdims=True)

                @pl.when(i == 2 * _NB - 1)
                def _fin():
                    out_ref[...] = out_ref[...] / jnp.maximum(cnt_ref[...], 1.0)

    frozen = lambda i: (jnp.minimum(i, _NB - 1), 0)
    const2 = lambda i: (0, 0)
    in_specs = [
        pl.BlockSpec((BR, D), frozen),
        pl.BlockSpec((_NC, BR, D), lambda i: (0, jnp.minimum(i, _NB - 1), 0)),
        pl.BlockSpec((D, D), const2),
        pl.BlockSpec((1, D), const2),
        pl.BlockSpec((D, D), const2),
        pl.BlockSpec((1, D), const2),
        pl.BlockSpec((1, D), const2),
        pl.BlockSpec((1, D), const2),
    ]
    args = [h, aggs, p['W1'], p['b1'].reshape(1, D), p['W2'], p['b2'].reshape(1, D),
            p['gamma'].reshape(1, D), p['beta'].reshape(1, D)]
    scratch = [
        pltpu.VMEM((N, D), jnp.float32),
        pltpu.VMEM((N if has_base else 8, D), jnp.float32),
        pltpu.VMEM((8, D), jnp.float32),
    ]
    if is_final:
        lin_W, lin_b, batch3 = final_args
        in_specs += [
            pl.BlockSpec((D, D), const2),
            pl.BlockSpec((1, D), const2),
            pl.BlockSpec((1, 1, BR), lambda i: (jnp.maximum(i - _NB, 0), 0, 0)),
        ]
        args += [lin_W, lin_b.reshape(1, D), batch3]
        out_spec = pl.BlockSpec((_G, D), const2)
        out_shape = jax.ShapeDtypeStruct((_G, D), jnp.float32)
        scratch.append(pltpu.VMEM((_G, D), jnp.float32))
    else:
        out_spec = pl.BlockSpec((BR, D), lambda i: (jnp.maximum(i - _NB, 0), 0))
        out_shape = jax.ShapeDtypeStruct((N, D), jnp.float32)

    return pl.pallas_call(
        body,
        grid=(2 * _NB,),
        in_specs=in_specs,
        out_specs=out_spec,
        out_shape=out_shape,
        scratch_shapes=scratch,
    )(*args)


def kernel(x, edge_index, batch, params):
    N, D = x.shape
    E = edge_index.shape[1]

    step = _CHUNK * _NSLOT
    per_w = -(-E // (_NW * step)) * step
    e_pad = _NW * per_w
    pad = e_pad - E
    ar = jnp.arange(pad, dtype=jnp.int32)
    fill_src = (ar * 97) % N
    fill_dst = N + (ar % _NPAD)
    src3 = jnp.concatenate([edge_index[0], fill_src]).reshape(_NW, per_w // _CHUNK, _CHUNK)
    dst3 = jnp.concatenate([edge_index[1], fill_dst]).reshape(_NW, per_w // _CHUNK, _CHUNK)
    batch3 = batch.reshape(_NB, 1, N // _NB)

    h = x
    convs = [params['conv1']] + list(params['convs'])
    for li, p in enumerate(convs):
        aggs = _sc_aggregate(h, src3, dst3)
        fin = (params['lin_W'], params['lin_b'], batch3) if li == len(convs) - 1 else None
        h = _tc_layer(h, aggs, p, has_base=li > 0, final_args=fin)
    return h


# speedup vs baseline: 1.7518x; 1.0404x over previous
import functools

import jax
import jax.numpy as jnp
from jax import lax
from jax.experimental import pallas as pl
from jax.experimental.pallas import tpu as pltpu
from jax.experimental.pallas import tpu_sc as plsc

_BN_EPS = 1e-5
_NC, _NS = 2, 16
_NW = _NC * _NS
_CHUNK = 64
_NSLOT = 5
_NPAD = 112
_G = 64
_NB = 5


def _sc_aggregate(h, src3, dst3):
    N, D = h.shape
    CH = src3.shape[1]
    mesh = plsc.VectorSubcoreMesh(core_axis_name="c", subcore_axis_name="s")

    NSLOT = _NSLOT
    assert CH % NSLOT == 0 and CH >= 2 * NSLOT

    @functools.partial(
        pl.kernel,
        out_type=jax.ShapeDtypeStruct((_NC, N + _NPAD, D), jnp.float32),
        mesh=mesh,
        scratch_types=[
            [pltpu.VMEM((_CHUNK, D), jnp.float32)] * NSLOT,
            [pltpu.VMEM((1, _CHUNK), jnp.int32)] * NSLOT,
            [pltpu.VMEM((1, _CHUNK), jnp.int32)] * NSLOT,
            pltpu.VMEM_SHARED((N + _NPAD, D), jnp.float32),
            [pltpu.SemaphoreType.DMA] * NSLOT,
            [pltpu.SemaphoreType.DMA] * NSLOT,
            [pltpu.SemaphoreType.DMA] * NSLOT,
            [pltpu.SemaphoreType.DMA] * NSLOT,
        ],
    )
    def agg_kernel(h_hbm, src_hbm, dst_hbm, out_hbm,
                   rows, sidx, didx, agg_sh, gsems, ssems, sisems, disems):
        c = lax.axis_index("c")
        s = lax.axis_index("s")
        wid = c * _NS + s

        def zrow(r, carry):
            for q in range(D // 16):
                rows[0][r, pl.ds(q * 16, 16)] = jnp.zeros((16,), jnp.float32)
            return carry

        lax.fori_loop(0, _CHUNK, zrow, 0)
        rz = (N + _NPAD) // _NS
        base_r = s * rz
        nfull, rem = rz // _CHUNK, rz % _CHUNK
        for q in range(nfull):
            pltpu.sync_copy(rows[0], agg_sh.at[pl.ds(base_r + q * _CHUNK, _CHUNK)])
        if rem:
            pltpu.sync_copy(rows[0].at[pl.ds(0, rem)],
                            agg_sh.at[pl.ds(base_r + nfull * _CHUNK, rem)])
        plsc.subcore_barrier()

        def si_start(k, sl):
            pltpu.async_copy(src_hbm.at[wid, pl.ds(k, 1)], sidx[sl], sisems[sl])

        def si_wait(k, sl):
            pltpu.make_async_copy(src_hbm.at[wid, pl.ds(k, 1)], sidx[sl], sisems[sl]).wait()

        def di_start(k, sl):
            pltpu.async_copy(dst_hbm.at[wid, pl.ds(k, 1)], didx[sl], disems[sl])

        def di_wait(k, sl):
            pltpu.make_async_copy(dst_hbm.at[wid, pl.ds(k, 1)], didx[sl], disems[sl]).wait()

        def g_start(k, sl):
            pltpu.async_copy(h_hbm.at[sidx[sl].at[0]], rows[sl], gsems[sl])

        def g_wait(k, sl):
            pltpu.make_async_copy(h_hbm.at[sidx[sl].at[0]], rows[sl], gsems[sl]).wait()

        def s_start(k, sl):
            pltpu.async_copy(rows[sl], agg_sh.at[didx[sl].at[0]], ssems[sl], add=True)

        def s_wait(k, sl):
            pltpu.make_async_copy(rows[sl], agg_sh.at[didx[sl].at[0]], ssems[sl]).wait()

        for k in range(NSLOT):
            si_start(k, k)
            di_start(k, k)
        for k in range(NSLOT - 1):
            si_wait(k, k)
            g_start(k, k)

        def body(jj, carry):
            j0 = jj * NSLOT
            for ph in range(NSLOT):
                j = j0 + ph
                g_wait(j, ph)

                @pl.when(j + NSLOT < CH)
                def _prefetch_src(j=j, ph=ph):
                    si_start(j + NSLOT, ph)

                di_wait(j, ph)
                s_start(j, ph)
                tgt = (ph + NSLOT - 1) % NSLOT

                @pl.when(j + NSLOT - 1 < CH)
                def _start_next(j=j, tgt=tgt):
                    @pl.when(j >= 1)
                    def _free_slot():
                        s_wait(j - 1, tgt)

                    si_wait(j + NSLOT - 1, tgt)
                    g_start(j + NSLOT - 1, tgt)

                    @pl.when(j >= 1)
                    def _prefetch_dst():
                        di_start(j + NSLOT - 1, tgt)

            return carry

        lax.fori_loop(0, CH // NSLOT, body, 0)
        for j in range(CH - NSLOT, CH):
            s_wait(j, j % NSLOT)
        plsc.subcore_barrier()
        pltpu.sync_copy(agg_sh.at[pl.ds(s * rz, rz)], out_hbm.at[c, pl.ds(s * rz, rz)])

    return agg_kernel(h, src3, dst3)


def _tc_layer(h, aggs, p, has_base, final_args=None):
    N, D = h.shape
    BR = N // _NB
    inv_n = 1.0 / N
    is_final = final_args is not None

    def body(*refs):
        if is_final:
            (h_ref, a_ref, W1_ref, b1_ref, W2_ref, b2_ref, g_ref, be_ref,
             lw_ref, lb_ref, bt_ref, out_ref,
             u2_all, h_all, st_ref, cnt_ref) = refs
        else:
            (h_ref, a_ref, W1_ref, b1_ref, W2_ref, b2_ref, g_ref, be_ref,
             out_ref, u2_all, h_all, st_ref) = refs
        i = pl.program_id(0)

        @pl.when(i < _NB)
        def _compute():
            u = h_ref[...] + a_ref[0] + a_ref[1]
            r1 = jnp.maximum(
                jnp.dot(u, W1_ref[...], preferred_element_type=jnp.float32)
                + b1_ref[...], 0.0)
            u2 = jnp.maximum(
                jnp.dot(r1, W2_ref[...], preferred_element_type=jnp.float32)
                + b2_ref[...], 0.0)
            u2_all[pl.ds(i * BR, BR), :] = u2
            if has_base:
                h_all[pl.ds(i * BR, BR), :] = h_ref[...]

            @pl.when(i == 0)
            def _init():
                st_ref[...] = jnp.zeros_like(st_ref)

            st_ref[0:1, :] += jnp.sum(u2, axis=0, keepdims=True)
            st_ref[1:2, :] += jnp.sum(u2 * u2, axis=0, keepdims=True)

        @pl.when(i >= _NB)
        def _normalize():
            blk = i - _NB
            u2 = u2_all[pl.ds(blk * BR, BR), :]
            mean = st_ref[0:1, :] * inv_n
            var = st_ref[1:2, :] * inv_n - mean * mean
            scale = g_ref[...] * lax.rsqrt(var + _BN_EPS)
            r = (u2 - mean) * scale + be_ref[...]
            if has_base:
                r = h_all[pl.ds(blk * BR, BR), :] + r
            if not is_final:
                out_ref[...] = r
            else:
                t = jnp.tanh(
                    jnp.dot(r, lw_ref[...], preferred_element_type=jnp.float32)
                    + lb_ref[...])
                seg = bt_ref[0]
                gids = lax.broadcasted_iota(jnp.int32, (_G, BR), 0)
                oh = (gids == seg).astype(jnp.float32)

                @pl.when(i == _NB)
                def _init_pool():
                    out_ref[...] = jnp.zeros_like(out_ref)
                    cnt_ref[...] = jnp.zeros_like(cnt_ref)

                out_ref[...] += jnp.dot(oh, t, preferred_element_type=jnp.float32)
                cnt_ref[...] += jnp.sum(oh, axis=1, keepdims=True)

                @pl.when(i == 2 * _NB - 1)
                def _fin():
                    out_ref[...] = out_ref[...] / jnp.maximum(cnt_ref[...], 1.0)

    frozen = lambda i: (jnp.minimum(i, _NB - 1), 0)
    const2 = lambda i: (0, 0)
    in_specs = [
        pl.BlockSpec((BR, D), frozen),
        pl.BlockSpec((_NC, BR, D), lambda i: (0, jnp.minimum(i, _NB - 1), 0)),
        pl.BlockSpec((D, D), const2),
        pl.BlockSpec((1, D), const2),
        pl.BlockSpec((D, D), const2),
        pl.BlockSpec((1, D), const2),
        pl.BlockSpec((1, D), const2),
        pl.BlockSpec((1, D), const2),
    ]
    args = [h, aggs, p['W1'], p['b1'].reshape(1, D), p['W2'], p['b2'].reshape(1, D),
            p['gamma'].reshape(1, D), p['beta'].reshape(1, D)]
    scratch = [
        pltpu.VMEM((N, D), jnp.float32),
        pltpu.VMEM((N if has_base else 8, D), jnp.float32),
        pltpu.VMEM((8, D), jnp.float32),
    ]
    if is_final:
        lin_W, lin_b, batch3 = final_args
        in_specs += [
            pl.BlockSpec((D, D), const2),
            pl.BlockSpec((1, D), const2),
            pl.BlockSpec((1, 1, BR), lambda i: (jnp.maximum(i - _NB, 0), 0, 0)),
        ]
        args += [lin_W, lin_b.reshape(1, D), batch3]
        out_spec = pl.BlockSpec((_G, D), const2)
        out_shape = jax.ShapeDtypeStruct((_G, D), jnp.float32)
        scratch.append(pltpu.VMEM((_G, D), jnp.float32))
    else:
        out_spec = pl.BlockSpec((BR, D), lambda i: (jnp.maximum(i - _NB, 0), 0))
        out_shape = jax.ShapeDtypeStruct((N, D), jnp.float32)

    return pl.pallas_call(
        body,
        grid=(2 * _NB,),
        in_specs=in_specs,
        out_specs=out_spec,
        out_shape=out_shape,
        scratch_shapes=scratch,
    )(*args)


def kernel(x, edge_index, batch, params):
    N, D = x.shape
    E = edge_index.shape[1]

    step = _CHUNK * _NSLOT
    per_w = -(-E // (_NW * step)) * step
    e_pad = _NW * per_w
    pad = e_pad - E
    ar = jnp.arange(pad, dtype=jnp.int32)
    fill_src = (ar * 97) % N
    fill_dst = N + (ar % _NPAD)
    src3 = jnp.concatenate([edge_index[0], fill_src]).reshape(_NW, per_w // _CHUNK, _CHUNK)
    dst3 = jnp.concatenate([edge_index[1], fill_dst]).reshape(_NW, per_w // _CHUNK, _CHUNK)
    batch3 = batch.reshape(_NB, 1, N // _NB)

    h = x
    convs = [params['conv1']] + list(params['convs'])
    for li, p in enumerate(convs):
        aggs = _sc_aggregate(h, src3, dst3)
        fin = (params['lin_W'], params['lin_b'], batch3) if li == len(convs) - 1 else None
        h = _tc_layer(h, aggs, p, has_base=li > 0, final_args=fin)
    return h


# speedup vs baseline: 1.7680x; 1.0092x over previous
import functools

import jax
import jax.numpy as jnp
from jax import lax
from jax.experimental import pallas as pl
from jax.experimental.pallas import tpu as pltpu
from jax.experimental.pallas import tpu_sc as plsc

_BN_EPS = 1e-5
_NC, _NS = 2, 16
_NW = _NC * _NS
_CHUNK = 64
_NSLOT = 5
_NPAD = 112
_G = 64
_NB = 2


def _sc_aggregate(h, src3, dst3):
    N, D = h.shape
    CH = src3.shape[1]
    mesh = plsc.VectorSubcoreMesh(core_axis_name="c", subcore_axis_name="s")

    NSLOT = _NSLOT
    assert CH % NSLOT == 0 and CH >= 2 * NSLOT

    @functools.partial(
        pl.kernel,
        out_type=jax.ShapeDtypeStruct((_NC, N + _NPAD, D), jnp.float32),
        mesh=mesh,
        scratch_types=[
            [pltpu.VMEM((_CHUNK, D), jnp.float32)] * NSLOT,
            [pltpu.VMEM((1, _CHUNK), jnp.int32)] * NSLOT,
            [pltpu.VMEM((1, _CHUNK), jnp.int32)] * NSLOT,
            pltpu.VMEM_SHARED((N + _NPAD, D), jnp.float32),
            [pltpu.SemaphoreType.DMA] * NSLOT,
            [pltpu.SemaphoreType.DMA] * NSLOT,
            [pltpu.SemaphoreType.DMA] * NSLOT,
            [pltpu.SemaphoreType.DMA] * NSLOT,
        ],
    )
    def agg_kernel(h_hbm, src_hbm, dst_hbm, out_hbm,
                   rows, sidx, didx, agg_sh, gsems, ssems, sisems, disems):
        c = lax.axis_index("c")
        s = lax.axis_index("s")
        wid = c * _NS + s

        def zrow(r, carry):
            for q in range(D // 16):
                rows[0][r, pl.ds(q * 16, 16)] = jnp.zeros((16,), jnp.float32)
            return carry

        lax.fori_loop(0, _CHUNK, zrow, 0)
        rz = (N + _NPAD) // _NS
        base_r = s * rz
        nfull, rem = rz // _CHUNK, rz % _CHUNK
        for q in range(nfull):
            pltpu.sync_copy(rows[0], agg_sh.at[pl.ds(base_r + q * _CHUNK, _CHUNK)])
        if rem:
            pltpu.sync_copy(rows[0].at[pl.ds(0, rem)],
                            agg_sh.at[pl.ds(base_r + nfull * _CHUNK, rem)])
        plsc.subcore_barrier()

        def si_start(k, sl):
            pltpu.async_copy(src_hbm.at[wid, pl.ds(k, 1)], sidx[sl], sisems[sl])

        def si_wait(k, sl):
            pltpu.make_async_copy(src_hbm.at[wid, pl.ds(k, 1)], sidx[sl], sisems[sl]).wait()

        def di_start(k, sl):
            pltpu.async_copy(dst_hbm.at[wid, pl.ds(k, 1)], didx[sl], disems[sl])

        def di_wait(k, sl):
            pltpu.make_async_copy(dst_hbm.at[wid, pl.ds(k, 1)], didx[sl], disems[sl]).wait()

        def g_start(k, sl):
            pltpu.async_copy(h_hbm.at[sidx[sl].at[0]], rows[sl], gsems[sl])

        def g_wait(k, sl):
            pltpu.make_async_copy(h_hbm.at[sidx[sl].at[0]], rows[sl], gsems[sl]).wait()

        def s_start(k, sl):
            pltpu.async_copy(rows[sl], agg_sh.at[didx[sl].at[0]], ssems[sl], add=True)

        def s_wait(k, sl):
            pltpu.make_async_copy(rows[sl], agg_sh.at[didx[sl].at[0]], ssems[sl]).wait()

        for k in range(NSLOT):
            si_start(k, k)
            di_start(k, k)
        for k in range(NSLOT - 1):
            si_wait(k, k)
            g_start(k, k)

        def body(jj, carry):
            j0 = jj * NSLOT
            for ph in range(NSLOT):
                j = j0 + ph
                g_wait(j, ph)

                @pl.when(j + NSLOT < CH)
                def _prefetch_src(j=j, ph=ph):
                    si_start(j + NSLOT, ph)

                di_wait(j, ph)
                s_start(j, ph)
                tgt = (ph + NSLOT - 1) % NSLOT

                @pl.when(j + NSLOT - 1 < CH)
                def _start_next(j=j, tgt=tgt):
                    @pl.when(j >= 1)
                    def _free_slot():
                        s_wait(j - 1, tgt)

                    si_wait(j + NSLOT - 1, tgt)
                    g_start(j + NSLOT - 1, tgt)

                    @pl.when(j >= 1)
                    def _prefetch_dst():
                        di_start(j + NSLOT - 1, tgt)

            return carry

        lax.fori_loop(0, CH // NSLOT, body, 0)
        for j in range(CH - NSLOT, CH):
            s_wait(j, j % NSLOT)
        plsc.subcore_barrier()
        pltpu.sync_copy(agg_sh.at[pl.ds(s * rz, rz)], out_hbm.at[c, pl.ds(s * rz, rz)])

    return agg_kernel(h, src3, dst3)


def _tc_layer(h, aggs, p, has_base, final_args=None):
    N, D = h.shape
    BR = N // _NB
    inv_n = 1.0 / N
    is_final = final_args is not None

    def body(*refs):
        if is_final:
            (h_ref, a_ref, W1_ref, b1_ref, W2_ref, b2_ref, g_ref, be_ref,
             lw_ref, lb_ref, bt_ref, out_ref,
             u2_all, h_all, st_ref, cnt_ref) = refs
        else:
            (h_ref, a_ref, W1_ref, b1_ref, W2_ref, b2_ref, g_ref, be_ref,
             out_ref, u2_all, h_all, st_ref) = refs
        i = pl.program_id(0)

        @pl.when(i < _NB)
        def _compute():
            u = h_ref[...] + a_ref[0] + a_ref[1]
            r1 = jnp.maximum(
                jnp.dot(u, W1_ref[...], preferred_element_type=jnp.float32)
                + b1_ref[...], 0.0)
            u2 = jnp.maximum(
                jnp.dot(r1, W2_ref[...], preferred_element_type=jnp.float32)
                + b2_ref[...], 0.0)
            u2_all[pl.ds(i * BR, BR), :] = u2
            if has_base:
                h_all[pl.ds(i * BR, BR), :] = h_ref[...]

            @pl.when(i == 0)
            def _init():
                st_ref[...] = jnp.zeros_like(st_ref)

            st_ref[0:1, :] += jnp.sum(u2, axis=0, keepdims=True)
            st_ref[1:2, :] += jnp.sum(u2 * u2, axis=0, keepdims=True)

        @pl.when(i >= _NB)
        def _normalize():
            blk = i - _NB
            u2 = u2_all[pl.ds(blk * BR, BR), :]
            mean = st_ref[0:1, :] * inv_n
            var = st_ref[1:2, :] * inv_n - mean * mean
            scale = g_ref[...] * lax.rsqrt(var + _BN_EPS)
            r = (u2 - mean) * scale + be_ref[...]
            if has_base:
                r = h_all[pl.ds(blk * BR, BR), :] + r
            if not is_final:
                out_ref[...] = r
            else:
                t = jnp.tanh(
                    jnp.dot(r, lw_ref[...], preferred_element_type=jnp.float32)
                    + lb_ref[...])
                seg = bt_ref[0]
                gids = lax.broadcasted_iota(jnp.int32, (_G, BR), 0)
                oh = (gids == seg).astype(jnp.float32)

                @pl.when(i == _NB)
                def _init_pool():
                    out_ref[...] = jnp.zeros_like(out_ref)
                    cnt_ref[...] = jnp.zeros_like(cnt_ref)

                out_ref[...] += jnp.dot(oh, t, preferred_element_type=jnp.float32)
                cnt_ref[...] += jnp.sum(oh, axis=1, keepdims=True)

                @pl.when(i == 2 * _NB - 1)
                def _fin():
                    out_ref[...] = out_ref[...] / jnp.maximum(cnt_ref[...], 1.0)

    frozen = lambda i: (jnp.minimum(i, _NB - 1), 0)
    const2 = lambda i: (0, 0)
    in_specs = [
        pl.BlockSpec((BR, D), frozen),
        pl.BlockSpec((_NC, BR, D), lambda i: (0, jnp.minimum(i, _NB - 1), 0)),
        pl.BlockSpec((D, D), const2),
        pl.BlockSpec((1, D), const2),
        pl.BlockSpec((D, D), const2),
        pl.BlockSpec((1, D), const2),
        pl.BlockSpec((1, D), const2),
        pl.BlockSpec((1, D), const2),
    ]
    args = [h, aggs, p['W1'], p['b1'].reshape(1, D), p['W2'], p['b2'].reshape(1, D),
            p['gamma'].reshape(1, D), p['beta'].reshape(1, D)]
    scratch = [
        pltpu.VMEM((N, D), jnp.float32),
        pltpu.VMEM((N if has_base else 8, D), jnp.float32),
        pltpu.VMEM((8, D), jnp.float32),
    ]
    if is_final:
        lin_W, lin_b, batch3 = final_args
        in_specs += [
            pl.BlockSpec((D, D), const2),
            pl.BlockSpec((1, D), const2),
            pl.BlockSpec((1, 1, BR), lambda i: (jnp.maximum(i - _NB, 0), 0, 0)),
        ]
        args += [lin_W, lin_b.reshape(1, D), batch3]
        out_spec = pl.BlockSpec((_G, D), const2)
        out_shape = jax.ShapeDtypeStruct((_G, D), jnp.float32)
        scratch.append(pltpu.VMEM((_G, D), jnp.float32))
    else:
        out_spec = pl.BlockSpec((BR, D), lambda i: (jnp.maximum(i - _NB, 0), 0))
        out_shape = jax.ShapeDtypeStruct((N, D), jnp.float32)

    return pl.pallas_call(
        body,
        grid=(2 * _NB,),
        in_specs=in_specs,
        out_specs=out_spec,
        out_shape=out_shape,
        scratch_shapes=scratch,
    )(*args)


def kernel(x, edge_index, batch, params):
    N, D = x.shape
    E = edge_index.shape[1]

    step = _CHUNK * _NSLOT
    per_w = -(-E // (_NW * step)) * step
    e_pad = _NW * per_w
    pad = e_pad - E
    ar = jnp.arange(pad, dtype=jnp.int32)
    fill_src = (ar * 97) % N
    fill_dst = N + (ar % _NPAD)
    src3 = jnp.concatenate([edge_index[0], fill_src]).reshape(_NW, per_w // _CHUNK, _CHUNK)
    dst3 = jnp.concatenate([edge_index[1], fill_dst]).reshape(_NW, per_w // _CHUNK, _CHUNK)
    batch3 = batch.reshape(_NB, 1, N // _NB)

    h = x
    convs = [params['conv1']] + list(params['convs'])
    for li, p in enumerate(convs):
        aggs = _sc_aggregate(h, src3, dst3)
        fin = (params['lin_W'], params['lin_b'], batch3) if li == len(convs) - 1 else None
        h = _tc_layer(h, aggs, p, has_base=li > 0, final_args=fin)
    return h
